# baseline (speedup 1.0000x reference)
import jax
import jax.numpy as jnp
from jax import lax
from jax.experimental import pallas as pl
from jax.experimental.pallas import tpu as pltpu
from jax.experimental.pallas import tpu_sc as plsc

_D_IN = 4096
_D_H = 2048
_NC = 2
_NS = 16
_L = 16
_COLS_W = 128
_UC = _COLS_W // _L
_ROWS_W = _D_IN // 2
_RCH = 256
_NRCH = _ROWS_W // _RCH
_RG = 8
_NPAIR = _NC * _NS // 2
_PK = plsc.PackFormat.INTERLEAVED


def _splat(v, i):
    idx = jnp.full((_L,), i, dtype=jnp.int32)
    return v.at[idx].get(mode="promise_in_bounds")


def _round_pair(x, y):
    return plsc.unpack(plsc.pack(x, y, format=_PK), format=_PK)


def _round8(v):
    v = list(v)
    for u in range(0, 8, 2):
        v[u], v[u + 1] = _round_pair(v[u], v[u + 1])
    return v


def _round_ref(ref, n):
    def body(k, _):
        x, y = _round_pair(ref[pl.ds(k * 2 * _L, _L)],
                           ref[pl.ds(k * 2 * _L + _L, _L)])
        ref[pl.ds(k * 2 * _L, _L)] = x
        ref[pl.ds(k * 2 * _L + _L, _L)] = y
        return 0

    lax.fori_loop(0, n // (2 * _L), body, 0)


def _sc_body(w_hbm, a_hbm, b_hbm, ws_hbm, be_hbm, out_hbm,
             buf0, buf1, a_v, b_v, ws1_v, ws2_v, be_v, pv_v,
             acc_v, mate_v, spx, sem0, sem1):
    cid = lax.axis_index("c")
    sid = lax.axis_index("s")
    pair = sid // 2
    half = sid % 2
    col0 = cid * (_NS // 2 * _COLS_W) + pair * _COLS_W
    row0 = half * _ROWS_W
    for c in range(2):
        pltpu.async_copy(
            w_hbm.at[pl.ds(row0 + c * _RCH, _RCH), pl.ds(col0, _COLS_W)],
            [buf0, buf1][c], [sem0, sem1][c])
    pltpu.sync_copy(a_hbm.at[pl.ds(row0, _ROWS_W)], a_v)
    pltpu.sync_copy(b_hbm.at[pl.ds(row0, _ROWS_W)], b_v)
    pltpu.sync_copy(ws_hbm.at[pl.ds(col0, _COLS_W)], ws1_v)
    pltpu.sync_copy(ws_hbm.at[pl.ds(_D_H + col0, _COLS_W)], ws2_v)
    pltpu.sync_copy(be_hbm.at[pl.ds(col0, _COLS_W)], be_v)
    _round_ref(a_v, _ROWS_W)
    _round_ref(b_v, _ROWS_W)
    _round_ref(ws1_v, _COLS_W)
    _round_ref(ws2_v, _COLS_W)

    zero = jnp.zeros((_L,), jnp.float32)

    def chunk_body(c, accs):
        def run(buf, sem, accs):
            pltpu.make_async_copy(
                w_hbm.at[pl.ds(0, _RCH), pl.ds(0, _COLS_W)], buf, sem).wait()

            def grp_body(g, carry, buf=buf):
                a1 = list(carry[:_UC])
                a2 = list(carry[_UC:])
                r0 = g * _RG
                base = c * _RCH + r0
                av = a_v[pl.ds(base, _L)]
                bv = b_v[pl.ds(base, _L)]
                for i in range(_RG):
                    ai = _splat(av, i)
                    bi = _splat(bv, i)
                    w = _round8([buf[r0 + i, pl.ds(u * _L, _L)]
                                 for u in range(_UC)])
                    for u in range(_UC):
                        a1[u] = a1[u] + ai * w[u]
                        a2[u] = a2[u] + bi * w[u]
                return tuple(a1) + tuple(a2)

            accs = lax.fori_loop(0, _RCH // _RG, grp_body, accs)

            @pl.when(c + 2 < _NRCH)
            def _():
                pltpu.async_copy(
                    w_hbm.at[pl.ds(row0 + (c + 2) * _RCH, _RCH),
                             pl.ds(col0, _COLS_W)],
                    buf, sem)

            return accs

        return lax.cond(c % 2 == 0,
                        lambda a: run(buf0, sem0, a),
                        lambda a: run(buf1, sem1, a),
                        accs)

    accs = lax.fori_loop(0, _NRCH, chunk_body, (zero,) * (2 * _UC))

    for k in range(2 * _UC):
        acc_v[pl.ds(k * _L, _L)] = accs[k]
    pltpu.sync_copy(acc_v, spx.at[sid])
    plsc.subcore_barrier()

    @pl.when(half == 0)
    def _():
        pltpu.sync_copy(spx.at[sid + 1], mate_v)
        ea = [acc_v[pl.ds(u * _L, _L)] + mate_v[pl.ds(u * _L, _L)]
              + be_v[pl.ds(u * _L, _L)] for u in range(_UC)]
        eb = [acc_v[pl.ds((_UC + u) * _L, _L)]
              + mate_v[pl.ds((_UC + u) * _L, _L)]
              + be_v[pl.ds(u * _L, _L)] for u in range(_UC)]
        ea = _round8(ea)
        eb = _round8(eb)
        tot = jnp.zeros((_L,), jnp.float32)
        for u in range(_UC):
            tot = tot + ea[u] * ws1_v[pl.ds(u * _L, _L)]
            tot = tot + eb[u] * ws2_v[pl.ds(u * _L, _L)]
        pv_v[...] = tot
        pltpu.sync_copy(pv_v, out_hbm.at[cid * (_NS // 2) + pair])


def kernel(a, b, W_embed, b_embed, W_scorer, b_scorer):
    mesh = plsc.VectorSubcoreMesh(core_axis_name="c", subcore_axis_name="s")
    run = pl.kernel(
        _sc_body,
        mesh=mesh,
        compiler_params=pltpu.CompilerParams(needs_layout_passes=False),
        out_type=jax.ShapeDtypeStruct((_NPAIR, _L), jnp.float32),
        scratch_types=[
            pltpu.VMEM((_RCH, _COLS_W), jnp.float32),
            pltpu.VMEM((_RCH, _COLS_W), jnp.float32),
            pltpu.VMEM((_ROWS_W,), jnp.float32),
            pltpu.VMEM((_ROWS_W,), jnp.float32),
            pltpu.VMEM((_COLS_W,), jnp.float32),
            pltpu.VMEM((_COLS_W,), jnp.float32),
            pltpu.VMEM((_COLS_W,), jnp.float32),
            pltpu.VMEM((_L,), jnp.float32),
            pltpu.VMEM((2 * _COLS_W,), jnp.float32),
            pltpu.VMEM((2 * _COLS_W,), jnp.float32),
            pltpu.VMEM_SHARED((_NS, 2 * _COLS_W), jnp.float32),
            pltpu.SemaphoreType.DMA,
            pltpu.SemaphoreType.DMA,
        ],
    )
    parts = run(W_embed, a, b, W_scorer.reshape(-1), b_embed)
    return jnp.sum(parts) + b_scorer[0]


# speedup vs baseline: 1.1512x; 1.1512x over previous
import jax
import jax.numpy as jnp
from jax import lax
from jax.experimental import pallas as pl
from jax.experimental.pallas import tpu as pltpu
from jax.experimental.pallas import tpu_sc as plsc

_D_IN = 4096
_D_H = 2048
_NC = 2
_NS = 16
_L = 16
_NW = _NC * _NS
_ROWS_W = _D_IN // _NW
_RCH = 16
_NRCH = _ROWS_W // _RCH
_SW = 8
_NSTRIP = _D_H // (_SW * _L)
_RG = 4


def _rtne_bf16_inplace(ref, n):
    def body(k, _):
        x = ref[pl.ds(k * _L, _L)]
        u = plsc.bitcast(x, jnp.int32)
        r = (u + 0x7FFF + ((u >> 16) & 1)) & jnp.int32(-65536)
        ref[pl.ds(k * _L, _L)] = plsc.bitcast(r, jnp.float32)
        return 0

    lax.fori_loop(0, n // _L, body, 0)


def _splat(v, lane):
    idx = jnp.full((_L,), lane, dtype=jnp.int32)
    return v.at[idx].get(mode="promise_in_bounds")


def _sc_body(w_hbm, a_hbm, b_hbm, ws_hbm, be_hbm, out_hbm,
             buf0, buf1, a_v, b_v, ws1_v, ws2_v, be_v, pv_v,
             sem0, sem1):
    cid = lax.axis_index("c")
    sid = lax.axis_index("s")
    wid = sid * _NC + cid
    row0 = wid * _ROWS_W
    bufs = [buf0, buf1]
    sems = [sem0, sem1]
    handles = [
        pltpu.async_copy(
            w_hbm.at[pl.ds(row0 + c * _RCH, _RCH), :], bufs[c], sems[c])
        for c in range(2)
    ]
    pltpu.sync_copy(a_hbm.at[pl.ds(row0, _ROWS_W)], a_v)
    pltpu.sync_copy(b_hbm.at[pl.ds(row0, _ROWS_W)], b_v)
    pltpu.sync_copy(ws_hbm.at[pl.ds(0, _D_H)], ws1_v)
    pltpu.sync_copy(ws_hbm.at[pl.ds(_D_H, _D_H)], ws2_v)
    pltpu.sync_copy(be_hbm, be_v)
    _rtne_bf16_inplace(a_v, _ROWS_W)
    _rtne_bf16_inplace(b_v, _ROWS_W)

    zero = jnp.zeros((_L,), jnp.float32)
    tots = (zero, zero)
    for c in range(_NRCH):
        handles[c % 2].wait()
        buf = bufs[c % 2]
        av = a_v[pl.ds(c * _RCH, _L)]
        bv = b_v[pl.ds(c * _RCH, _L)]

        def strip_body(t, tt, buf=buf, av=av, bv=bv):
            col0 = t * (_SW * _L)
            ws1c = [ws1_v[pl.ds(col0 + u * _L, _L)] for u in range(_SW)]
            ws2c = [ws2_v[pl.ds(col0 + u * _L, _L)] for u in range(_SW)]

            def row_body(g, rr, buf=buf):
                r1, r2 = rr
                for i in range(_RG):
                    lane = g * _RG + i
                    ai = _splat(av, lane)
                    bi = _splat(bv, lane)
                    w = [buf[lane, pl.ds(col0 + u * _L, _L)]
                         for u in range(_SW)]
                    s1 = w[0] * ws1c[0]
                    s2 = w[0] * ws2c[0]
                    for u in range(1, _SW):
                        s1 = s1 + w[u] * ws1c[u]
                        s2 = s2 + w[u] * ws2c[u]
                    r1 = r1 + ai * s1
                    r2 = r2 + bi * s2
                return (r1, r2)

            return lax.fori_loop(0, _RCH // _RG, row_body, tt)

        tots = lax.fori_loop(0, _NSTRIP, strip_body, tots)
        if c + 2 < _NRCH:
            handles[c % 2] = pltpu.async_copy(
                w_hbm.at[pl.ds(row0 + (c + 2) * _RCH, _RCH), :],
                bufs[c % 2], sems[c % 2])

    pv_v[...] = tots[0] + tots[1]

    @pl.when(wid == 0)
    def _():
        def bias_body(k, bv_):
            c0 = k * _L
            return bv_ + be_v[pl.ds(c0, _L)] * (
                ws1_v[pl.ds(c0, _L)] + ws2_v[pl.ds(c0, _L)])

        bias_v = lax.fori_loop(0, _D_H // _L, bias_body, zero)
        pv_v[...] = pv_v[...] + bias_v

    pltpu.sync_copy(pv_v, out_hbm.at[wid])


def kernel(a, b, W_embed, b_embed, W_scorer, b_scorer):
    mesh = plsc.VectorSubcoreMesh(core_axis_name="c", subcore_axis_name="s")
    run = pl.kernel(
        _sc_body,
        mesh=mesh,
        compiler_params=pltpu.CompilerParams(needs_layout_passes=False),
        out_type=jax.ShapeDtypeStruct((_NW, _L), jnp.float32),
        scratch_types=[
            pltpu.VMEM((_RCH, _D_H), jnp.float32),
            pltpu.VMEM((_RCH, _D_H), jnp.float32),
            pltpu.VMEM((_ROWS_W,), jnp.float32),
            pltpu.VMEM((_ROWS_W,), jnp.float32),
            pltpu.VMEM((_D_H,), jnp.float32),
            pltpu.VMEM((_D_H,), jnp.float32),
            pltpu.VMEM((_D_H,), jnp.float32),
            pltpu.VMEM((_L,), jnp.float32),
            pltpu.SemaphoreType.DMA,
            pltpu.SemaphoreType.DMA,
        ],
    )
    parts = run(W_embed, a, b, W_scorer.reshape(-1), b_embed)
    return jnp.sum(parts) + b_scorer[0]


# speedup vs baseline: 1.2209x; 1.0606x over previous
import jax
import jax.numpy as jnp
from jax import lax
from jax.experimental import pallas as pl
from jax.experimental.pallas import tpu as pltpu
from jax.experimental.pallas import tpu_sc as plsc

_D_IN = 4096
_D_H = 2048
_NC = 2
_NS = 16
_L = 16
_NW = _NC * _NS
_ROWS_W = _D_IN // _NW
_RCH = 16
_NRCH = _ROWS_W // _RCH
_SW = 8
_NSTRIP = _D_H // (_SW * _L)
_RG = 4


def _rtne_bf16_inplace(ref, n):
    def body(k, _):
        x = ref[pl.ds(k * _L, _L)]
        u = plsc.bitcast(x, jnp.int32)
        r = (u + 0x7FFF + ((u >> 16) & 1)) & jnp.int32(-65536)
        ref[pl.ds(k * _L, _L)] = plsc.bitcast(r, jnp.float32)
        return 0

    lax.fori_loop(0, n // _L, body, 0)


def _splat(v, lane):
    idx = jnp.full((_L,), lane, dtype=jnp.int32)
    return v.at[idx].get(mode="promise_in_bounds")


def _sc_body(w_hbm, a_hbm, b_hbm, ws_hbm, be_hbm, out_hbm,
             buf0, buf1, a_v, b_v, ws1_v, ws2_v, be_v, pv_v,
             sem0, sem1, sem2):
    cid = lax.axis_index("c")
    sid = lax.axis_index("s")
    wid = sid * _NC + cid
    row0 = wid * _ROWS_W
    bufs = [buf0, buf1]
    sems = [sem0, sem1]
    handles = [
        pltpu.async_copy(
            w_hbm.at[pl.ds(row0 + c * _RCH, _RCH), :], bufs[c], sems[c])
        for c in range(2)
    ]
    small = [
        pltpu.async_copy(a_hbm.at[pl.ds(row0, _ROWS_W)], a_v, sem2),
        pltpu.async_copy(b_hbm.at[pl.ds(row0, _ROWS_W)], b_v, sem2),
        pltpu.async_copy(ws_hbm.at[pl.ds(0, _D_H)], ws1_v, sem2),
        pltpu.async_copy(ws_hbm.at[pl.ds(_D_H, _D_H)], ws2_v, sem2),
        pltpu.async_copy(be_hbm, be_v, sem2),
    ]
    for h in small:
        h.wait()
    _rtne_bf16_inplace(a_v, _ROWS_W)
    _rtne_bf16_inplace(b_v, _ROWS_W)

    zero = jnp.zeros((_L,), jnp.float32)
    tots = (zero, zero)
    for c in range(_NRCH):
        handles[c % 2].wait()
        buf = bufs[c % 2]
        av = a_v[pl.ds(c * _RCH, _L)]
        bv = b_v[pl.ds(c * _RCH, _L)]

        def strip_body(t, tt, buf=buf, av=av, bv=bv):
            t1, t2 = tt
            col0 = t * (_SW * _L)
            zero_ = jnp.zeros((_L,), jnp.float32)

            def row_body(g, accs, buf=buf):
                a1 = list(accs[:_SW])
                a2 = list(accs[_SW:])
                for i in range(_RG):
                    lane = g * _RG + i
                    ai = _splat(av, lane)
                    bi = _splat(bv, lane)
                    for u in range(_SW):
                        w = buf[lane, pl.ds(col0 + u * _L, _L)]
                        a1[u] = a1[u] + ai * w
                        a2[u] = a2[u] + bi * w
                return tuple(a1) + tuple(a2)

            accs = lax.fori_loop(0, _RCH // _RG, row_body,
                                 (zero_,) * (2 * _SW))
            for u in range(_SW):
                t1 = t1 + accs[u] * ws1_v[pl.ds(col0 + u * _L, _L)]
                t2 = t2 + accs[_SW + u] * ws2_v[pl.ds(col0 + u * _L, _L)]
            return (t1, t2)

        tots = lax.fori_loop(0, _NSTRIP, strip_body, tots)
        if c + 2 < _NRCH:
            handles[c % 2] = pltpu.async_copy(
                w_hbm.at[pl.ds(row0 + (c + 2) * _RCH, _RCH), :],
                bufs[c % 2], sems[c % 2])

    pv_v[...] = tots[0] + tots[1]

    @pl.when(wid == 0)
    def _():
        def bias_body(k, bv_):
            c0 = k * _L
            return bv_ + be_v[pl.ds(c0, _L)] * (
                ws1_v[pl.ds(c0, _L)] + ws2_v[pl.ds(c0, _L)])

        bias_v = lax.fori_loop(0, _D_H // _L, bias_body, zero)
        pv_v[...] = pv_v[...] + bias_v

    pltpu.sync_copy(pv_v, out_hbm.at[wid])


def kernel(a, b, W_embed, b_embed, W_scorer, b_scorer):
    mesh = plsc.VectorSubcoreMesh(core_axis_name="c", subcore_axis_name="s")
    run = pl.kernel(
        _sc_body,
        mesh=mesh,
        compiler_params=pltpu.CompilerParams(needs_layout_passes=False),
        out_type=jax.ShapeDtypeStruct((_NW, _L), jnp.float32),
        scratch_types=[
            pltpu.VMEM((_RCH, _D_H), jnp.float32),
            pltpu.VMEM((_RCH, _D_H), jnp.float32),
            pltpu.VMEM((_ROWS_W,), jnp.float32),
            pltpu.VMEM((_ROWS_W,), jnp.float32),
            pltpu.VMEM((_D_H,), jnp.float32),
            pltpu.VMEM((_D_H,), jnp.float32),
            pltpu.VMEM((_D_H,), jnp.float32),
            pltpu.VMEM((_L,), jnp.float32),
            pltpu.SemaphoreType.DMA,
            pltpu.SemaphoreType.DMA,
            pltpu.SemaphoreType.DMA,
        ],
    )
    parts = run(W_embed, a, b, W_scorer.reshape(-1), b_embed)
    return jnp.sum(parts) + b_scorer[0]


# speedup vs baseline: 1.2903x; 1.0568x over previous
import jax
import jax.numpy as jnp
from jax import lax
from jax.experimental import pallas as pl
from jax.experimental.pallas import tpu as pltpu
from jax.experimental.pallas import tpu_sc as plsc

_D_IN = 4096
_D_H = 2048
_NC = 2
_NS = 16
_L = 16
_NW = _NC * _NS
_ROWS_W = _D_IN // _NW
_RCH = 16
_NRCH = _ROWS_W // _RCH
_SW = 8
_NSTRIP = _D_H // (_SW * _L)
_RG = 8


def _rtne_bf16_inplace(ref, n):
    def body(k, _):
        x = ref[pl.ds(k * _L, _L)]
        u = plsc.bitcast(x, jnp.int32)
        r = (u + 0x7FFF + ((u >> 16) & 1)) & jnp.int32(-65536)
        ref[pl.ds(k * _L, _L)] = plsc.bitcast(r, jnp.float32)
        return 0

    lax.fori_loop(0, n // _L, body, 0)


def _splat(v, lane):
    idx = jnp.full((_L,), lane, dtype=jnp.int32)
    return v.at[idx].get(mode="promise_in_bounds")


def _sc_body(w_hbm, a_hbm, b_hbm, ws_hbm, be_hbm, out_hbm,
             buf0, buf1, a_v, b_v, ws1_v, ws2_v, be_v, pv_v,
             sem0, sem1, sem2):
    cid = lax.axis_index("c")
    sid = lax.axis_index("s")
    wid = sid * _NC + cid
    row0 = wid * _ROWS_W
    bufs = [buf0, buf1]
    sems = [sem0, sem1]
    handles = [
        pltpu.async_copy(
            w_hbm.at[pl.ds(row0 + c * _RCH, _RCH), :], bufs[c], sems[c])
        for c in range(2)
    ]
    small = [
        pltpu.async_copy(a_hbm.at[pl.ds(row0, _ROWS_W)], a_v, sem2),
        pltpu.async_copy(b_hbm.at[pl.ds(row0, _ROWS_W)], b_v, sem2),
        pltpu.async_copy(ws_hbm.at[pl.ds(0, _D_H)], ws1_v, sem2),
        pltpu.async_copy(ws_hbm.at[pl.ds(_D_H, _D_H)], ws2_v, sem2),
        pltpu.async_copy(be_hbm, be_v, sem2),
    ]
    for h in small:
        h.wait()
    _rtne_bf16_inplace(a_v, _ROWS_W)
    _rtne_bf16_inplace(b_v, _ROWS_W)

    zero = jnp.zeros((_L,), jnp.float32)
    tots = (zero, zero)
    for c in range(_NRCH):
        handles[c % 2].wait()
        buf = bufs[c % 2]
        av = a_v[pl.ds(c * _RCH, _L)]
        bv = b_v[pl.ds(c * _RCH, _L)]

        def strip_body(t, tt, buf=buf, av=av, bv=bv):
            t1, t2 = tt
            col0 = t * (_SW * _L)
            zero_ = jnp.zeros((_L,), jnp.float32)

            def row_body(g, accs, buf=buf):
                a1 = list(accs[:_SW])
                a2 = list(accs[_SW:])
                for i in range(_RG):
                    lane = g * _RG + i
                    ai = _splat(av, lane)
                    bi = _splat(bv, lane)
                    for u in range(_SW):
                        w = buf[lane, pl.ds(col0 + u * _L, _L)]
                        a1[u] = a1[u] + ai * w
                        a2[u] = a2[u] + bi * w
                return tuple(a1) + tuple(a2)

            accs = lax.fori_loop(0, _RCH // _RG, row_body,
                                 (zero_,) * (2 * _SW))
            for u in range(_SW):
                t1 = t1 + accs[u] * ws1_v[pl.ds(col0 + u * _L, _L)]
                t2 = t2 + accs[_SW + u] * ws2_v[pl.ds(col0 + u * _L, _L)]
            return (t1, t2)

        tots = lax.fori_loop(0, _NSTRIP, strip_body, tots)
        if c + 2 < _NRCH:
            handles[c % 2] = pltpu.async_copy(
                w_hbm.at[pl.ds(row0 + (c + 2) * _RCH, _RCH), :],
                bufs[c % 2], sems[c % 2])

    pv_v[...] = tots[0] + tots[1]

    @pl.when(wid == 0)
    def _():
        def bias_body(k, bv_):
            c0 = k * _L
            return bv_ + be_v[pl.ds(c0, _L)] * (
                ws1_v[pl.ds(c0, _L)] + ws2_v[pl.ds(c0, _L)])

        bias_v = lax.fori_loop(0, _D_H // _L, bias_body, zero)
        pv_v[...] = pv_v[...] + bias_v

    pltpu.sync_copy(pv_v, out_hbm.at[wid])


def kernel(a, b, W_embed, b_embed, W_scorer, b_scorer):
    mesh = plsc.VectorSubcoreMesh(core_axis_name="c", subcore_axis_name="s")
    run = pl.kernel(
        _sc_body,
        mesh=mesh,
        compiler_params=pltpu.CompilerParams(needs_layout_passes=False),
        out_type=jax.ShapeDtypeStruct((_NW, _L), jnp.float32),
        scratch_types=[
            pltpu.VMEM((_RCH, _D_H), jnp.float32),
            pltpu.VMEM((_RCH, _D_H), jnp.float32),
            pltpu.VMEM((_ROWS_W,), jnp.float32),
            pltpu.VMEM((_ROWS_W,), jnp.float32),
            pltpu.VMEM((_D_H,), jnp.float32),
            pltpu.VMEM((_D_H,), jnp.float32),
            pltpu.VMEM((_D_H,), jnp.float32),
            pltpu.VMEM((_L,), jnp.float32),
            pltpu.SemaphoreType.DMA,
            pltpu.SemaphoreType.DMA,
            pltpu.SemaphoreType.DMA,
        ],
    )
    parts = run(W_embed, a, b, W_scorer.reshape(-1), b_embed)
    return jnp.sum(parts) + b_scorer[0]
